# baseline (speedup 1.0000x reference)
import functools
import jax
import jax.numpy as jnp
from jax import lax
from jax.experimental import pallas as pl
from jax.experimental.pallas import tpu as pltpu
from jax.experimental.pallas import tpu_sc as plsc

_N, _D = 1_000_000, 3
_NW = 32
_CHR = 5208
_NCH = 6
_RPW = _CHR * _NCH
_RTAIL_BASE = _NW * _RPW
_RTAIL = _N - _RTAIL_BASE


def _sc_body(x_hbm, u0_hbm, u2_hbm, out_hbm, buf, hb):
    wid = lax.axis_index("s") * 2 + lax.axis_index("c")
    base = wid * _RPW
    for k in range(_NCH):
        b = base + k * _CHR
        pltpu.sync_copy(x_hbm.at[pl.ds(b, _CHR)], buf)
        pltpu.sync_copy(buf, out_hbm.at[pl.ds(b, _CHR)])

    @pl.when(wid == _NW - 1)
    def _():
        pltpu.sync_copy(
            x_hbm.at[pl.ds(_RTAIL_BASE, _RTAIL)], buf.at[pl.ds(0, _RTAIL)]
        )
        pltpu.sync_copy(
            buf.at[pl.ds(0, _RTAIL)], out_hbm.at[pl.ds(_RTAIL_BASE, _RTAIL)]
        )

    @pl.when(wid == 0)
    def _():
        pltpu.sync_copy(x_hbm.at[pl.ds(0, 8)], hb)
        pltpu.sync_copy(u0_hbm, hb.at[pl.ds(0, 1)])
        pltpu.sync_copy(u2_hbm, hb.at[pl.ds(2, 1)])
        pltpu.sync_copy(hb, out_hbm.at[pl.ds(0, 8)])


def kernel(x):
    u0 = jnp.array([[10.0, 11.0, 12.0]], dtype=jnp.float32)
    u2 = jnp.array([[20.0, 21.0, 22.0]], dtype=jnp.float32)
    mesh = plsc.VectorSubcoreMesh(core_axis_name="c", subcore_axis_name="s")
    k = functools.partial(
        pl.kernel,
        out_type=jax.ShapeDtypeStruct((_N, _D), jnp.float32),
        mesh=mesh,
        scratch_types=[
            pltpu.VMEM((_CHR, _D), jnp.float32),
            pltpu.VMEM((8, _D), jnp.float32),
        ],
        compiler_params=pltpu.CompilerParams(use_tc_tiling_on_sc=False),
    )(_sc_body)
    return k(x, u0, u2)


# speedup vs baseline: 4.7951x; 4.7951x over previous
import functools
import jax
import jax.numpy as jnp
from jax import lax
from jax.experimental import pallas as pl
from jax.experimental.pallas import tpu as pltpu
from jax.experimental.pallas import tpu_sc as plsc

_N, _D = 1_000_000, 3
_NW = 32
_CHR = 1008
_NCH = 31
_RPW = _CHR * _NCH
_RTAIL_BASE = _NW * _RPW
_RTAIL = _N - _RTAIL_BASE


def _sc_body(x_hbm, u0_hbm, u2_hbm, out_hbm, buf, hb):
    wid = lax.axis_index("s") * 2 + lax.axis_index("c")
    base = wid * _RPW
    for k in range(_NCH):
        b = base + k * _CHR
        pltpu.sync_copy(x_hbm.at[pl.ds(b, _CHR)], buf)
        pltpu.sync_copy(buf, out_hbm.at[pl.ds(b, _CHR)])

    @pl.when(wid == _NW - 1)
    def _():
        pltpu.sync_copy(
            x_hbm.at[pl.ds(_RTAIL_BASE, _RTAIL)], buf.at[pl.ds(0, _RTAIL)]
        )
        pltpu.sync_copy(
            buf.at[pl.ds(0, _RTAIL)], out_hbm.at[pl.ds(_RTAIL_BASE, _RTAIL)]
        )

    @pl.when(wid == 0)
    def _():
        pltpu.sync_copy(x_hbm.at[pl.ds(0, 8)], hb)
        pltpu.sync_copy(u0_hbm, hb.at[pl.ds(0, 1)])
        pltpu.sync_copy(u2_hbm, hb.at[pl.ds(2, 1)])
        pltpu.sync_copy(hb, out_hbm.at[pl.ds(0, 8)])


def kernel(x):
    u0 = jnp.array([[10.0, 11.0, 12.0]], dtype=jnp.float32)
    u2 = jnp.array([[20.0, 21.0, 22.0]], dtype=jnp.float32)
    mesh = plsc.VectorSubcoreMesh(core_axis_name="c", subcore_axis_name="s")
    k = functools.partial(
        pl.kernel,
        out_type=jax.ShapeDtypeStruct((_N, _D), jnp.float32),
        mesh=mesh,
        scratch_types=[
            pltpu.VMEM((_CHR, _D), jnp.float32),
            pltpu.VMEM((8, _D), jnp.float32),
        ],
    )(_sc_body)
    return k(x, u0, u2)


# speedup vs baseline: 232.1826x; 48.4203x over previous
import jax
import jax.numpy as jnp
from jax.experimental import pallas as pl

_N, _D = 1_000_000, 3
_B = 65536
_GRID = -(-_N // _B)


def _copy_body(xt_ref, ot_ref):
    vals = xt_ref[...]

    @pl.when(pl.program_id(0) == 0)
    def _():
        r = jax.lax.broadcasted_iota(jnp.int32, (_D, _B), 0).astype(jnp.float32)
        c = jax.lax.broadcasted_iota(jnp.int32, (_D, _B), 1)
        patched = jnp.where(c == 0, 10.0 + r, jnp.where(c == 2, 20.0 + r, vals))
        ot_ref[...] = patched

    @pl.when(pl.program_id(0) != 0)
    def _():
        ot_ref[...] = vals


def kernel(x):
    xt = jnp.swapaxes(x, 0, 1)
    out_t = pl.pallas_call(
        _copy_body,
        grid=(_GRID,),
        in_specs=[pl.BlockSpec((_D, _B), lambda i: (0, i))],
        out_specs=pl.BlockSpec((_D, _B), lambda i: (0, i)),
        out_shape=jax.ShapeDtypeStruct((_D, _N), jnp.float32),
    )(xt)
    return jnp.swapaxes(out_t, 0, 1)


# speedup vs baseline: 320.3330x; 1.3797x over previous
import jax
import jax.numpy as jnp
from jax.experimental import pallas as pl

_N, _D = 1_000_000, 3
_B = 131072
_GRID = -(-_N // _B)


def _copy_body(xt_ref, ot_ref):
    vals = xt_ref[...]

    @pl.when(pl.program_id(0) == 0)
    def _():
        r = jax.lax.broadcasted_iota(jnp.int32, (_D, _B), 0).astype(jnp.float32)
        c = jax.lax.broadcasted_iota(jnp.int32, (_D, _B), 1)
        patched = jnp.where(c == 0, 10.0 + r, jnp.where(c == 2, 20.0 + r, vals))
        ot_ref[...] = patched

    @pl.when(pl.program_id(0) != 0)
    def _():
        ot_ref[...] = vals


def kernel(x):
    xt = jnp.swapaxes(x, 0, 1)
    out_t = pl.pallas_call(
        _copy_body,
        grid=(_GRID,),
        in_specs=[pl.BlockSpec((_D, _B), lambda i: (0, i))],
        out_specs=pl.BlockSpec((_D, _B), lambda i: (0, i)),
        out_shape=jax.ShapeDtypeStruct((_D, _N), jnp.float32),
    )(xt)
    return jnp.swapaxes(out_t, 0, 1)


# speedup vs baseline: 342.8314x; 1.0702x over previous
import jax
import jax.numpy as jnp
from jax.experimental import pallas as pl

_N, _D = 1_000_000, 3
_B = 262144
_GRID = -(-_N // _B)


def _copy_body(xt_ref, ot_ref):
    vals = xt_ref[...]

    @pl.when(pl.program_id(0) == 0)
    def _():
        r = jax.lax.broadcasted_iota(jnp.int32, (_D, _B), 0).astype(jnp.float32)
        c = jax.lax.broadcasted_iota(jnp.int32, (_D, _B), 1)
        patched = jnp.where(c == 0, 10.0 + r, jnp.where(c == 2, 20.0 + r, vals))
        ot_ref[...] = patched

    @pl.when(pl.program_id(0) != 0)
    def _():
        ot_ref[...] = vals


def kernel(x):
    xt = jnp.swapaxes(x, 0, 1)
    out_t = pl.pallas_call(
        _copy_body,
        grid=(_GRID,),
        in_specs=[pl.BlockSpec((_D, _B), lambda i: (0, i))],
        out_specs=pl.BlockSpec((_D, _B), lambda i: (0, i)),
        out_shape=jax.ShapeDtypeStruct((_D, _N), jnp.float32),
    )(xt)
    return jnp.swapaxes(out_t, 0, 1)
